# Initial kernel scaffold; baseline (speedup 1.0000x reference)
import jax
import jax.numpy as jnp
from jax.experimental import pallas as pl


def kernel(x, labels, Wm1, bm1, Wm2, bm2, Wm3, bm3, Wv1, bv1, Wv2, bv2, Wv3, bv3):
    raise NotImplementedError("write your pallas kernel here")


# speedup vs baseline: 12.3182x; 12.3182x over previous
import functools

import jax
import jax.numpy as jnp
from jax.experimental import pallas as pl
from jax.experimental.pallas import tpu as pltpu

B, S, D, K = 16, 4096, 128, 10
KP = 16


def _tc_body(x_ref, lab_ref, Wm1, bm1, Wm2, bm2, Wm3, bm3,
             Wv1, bv1, Wv2, bv2, Wv3, bv3, out_ref):
    lab = lab_ref[0, 0, :]
    ks = jax.lax.broadcasted_iota(jnp.int32, (KP, S), 0)
    oh = (ks == lab[None, :]).astype(jnp.float32)
    seg = jnp.dot(oh, x_ref[0], preferred_element_type=jnp.float32)
    cnt = jnp.sum(oh, axis=1, keepdims=True)
    cc = seg / jnp.maximum(cnt, 1e-30)

    def mlp(h, W1, b1, W2, b2, W3, b3):
        h = jax.nn.relu(jnp.dot(h, W1[...], preferred_element_type=jnp.float32) + b1[...])
        h = jax.nn.relu(jnp.dot(h, W2[...], preferred_element_type=jnp.float32) + b2[...])
        h = jax.nn.sigmoid(jnp.dot(h, W3[...], preferred_element_type=jnp.float32) + b3[...])
        return h * 2.0 - 1.0

    out_ref[0, 0] = mlp(cc, Wm1, bm1, Wm2, bm2, Wm3, bm3)
    out_ref[1, 0] = mlp(cc, Wv1, bv1, Wv2, bv2, Wv3, bv3)


@functools.partial(jax.jit, static_argnames=("interpret",))
def _run(x, labels, Wm1, bm1, Wm2, bm2, Wm3, bm3,
         Wv1, bv1, Wv2, bv2, Wv3, bv3, interpret=False):
    lab3 = labels.astype(jnp.int32).reshape(B, 1, S)
    wspec = pl.BlockSpec((D, D), lambda b: (0, 0))
    bspec = pl.BlockSpec((D,), lambda b: (0,))
    out = pl.pallas_call(
        _tc_body,
        grid=(B,),
        in_specs=[
            pl.BlockSpec((1, S, D), lambda b: (b, 0, 0)),
            pl.BlockSpec((1, 1, S), lambda b: (b, 0, 0)),
            wspec, bspec, wspec, bspec, wspec, bspec,
            wspec, bspec, wspec, bspec, wspec, bspec,
        ],
        out_specs=pl.BlockSpec((2, 1, KP, D), lambda b: (0, b, 0, 0)),
        out_shape=jax.ShapeDtypeStruct((2, B, KP, D), jnp.float32),
        interpret=interpret,
    )(x, lab3, Wm1, bm1, Wm2, bm2, Wm3, bm3, Wv1, bv1, Wv2, bv2, Wv3, bv3)
    return out[:, :, :K, :]


def kernel(x, labels, Wm1, bm1, Wm2, bm2, Wm3, bm3,
           Wv1, bv1, Wv2, bv2, Wv3, bv3):
    return _run(x, labels, Wm1, bm1, Wm2, bm2, Wm3, bm3,
                Wv1, bv1, Wv2, bv2, Wv3, bv3)
